# baseline (speedup 1.0000x reference)
import functools

import jax
import jax.numpy as jnp
from jax import lax
from jax.experimental import pallas as pl
from jax.experimental.pallas import tpu as pltpu
from jax.experimental.pallas import tpu_sc as plsc

HIDDEN = 768
NUM_EXPERTS = 8
TOP_K = 2

CHUNK = 1024
NBUF = 6
NW = 32


def _gate_body(x_hbm, wt_ref, logits_ref, xbuf, sem):
    i = pl.program_id(0)
    nchunk = pl.num_programs(0)
    slot = jax.lax.rem(i, NBUF)

    def start(c, s):
        pltpu.make_async_copy(
            x_hbm.at[pl.ds(c * CHUNK, CHUNK)], xbuf.at[s], sem.at[s]
        ).start()

    @pl.when(i == 0)
    def _():
        for b in range(NBUF):
            start(b, b)

    pltpu.make_async_copy(
        x_hbm.at[pl.ds(i * CHUNK, CHUNK)], xbuf.at[slot], sem.at[slot]
    ).wait()

    xb = xbuf[slot]
    logits = jnp.dot(xb, wt_ref[...], preferred_element_type=jnp.float32)
    logits_ref[0] = logits.T

    @pl.when(i + NBUF < nchunk)
    def _():
        start(i + NBUF, slot)


def _sc_topk(lg_hbm, i1_hbm, i2_hbm, w1_hbm, w2_hbm,
             lbuf, i1b, i2b, w1b, w2b):
    wid = lax.axis_index("s") * 2 + lax.axis_index("c")
    pltpu.sync_copy(lg_hbm.at[wid], lbuf)

    def group(g, carry):
        b16 = g * 16
        le = [lbuf[e, pl.ds(b16, 16)] for e in range(NUM_EXPERTS)]
        m1 = le[0]
        i1 = jnp.zeros((16,), jnp.int32)
        for e in range(1, NUM_EXPERTS):
            c = le[e] > m1
            m1 = jnp.where(c, le[e], m1)
            i1 = jnp.where(c, e, i1)
        m2 = jnp.full((16,), -1e30, jnp.float32)
        i2 = jnp.zeros((16,), jnp.int32)
        for e in range(NUM_EXPERTS):
            c = jnp.logical_and(le[e] > m2, i1 != e)
            m2 = jnp.where(c, le[e], m2)
            i2 = jnp.where(c, e, i2)
        ex = jnp.exp(m2 - m1)
        w1 = 1.0 / (1.0 + ex)
        i1b[pl.ds(b16, 16)] = i1
        i2b[pl.ds(b16, 16)] = i2
        w1b[pl.ds(b16, 16)] = w1
        w2b[pl.ds(b16, 16)] = 1.0 - w1
        return carry

    lax.fori_loop(0, CHUNK // 16, group, 0)

    base = wid * CHUNK
    pltpu.sync_copy(i1b, i1_hbm.at[pl.ds(base, CHUNK)])
    pltpu.sync_copy(i2b, i2_hbm.at[pl.ds(base, CHUNK)])
    pltpu.sync_copy(w1b, w1_hbm.at[pl.ds(base, CHUNK)])
    pltpu.sync_copy(w2b, w2_hbm.at[pl.ds(base, CHUNK)])


@jax.jit
def kernel(x, W):
    b, s, h = x.shape
    n = b * s
    x_flat = x.reshape(n, h)
    wt = W.T
    lg = pl.pallas_call(
        _gate_body,
        grid=(n // CHUNK,),
        in_specs=[
            pl.BlockSpec(memory_space=pltpu.HBM),
            pl.BlockSpec((h, NUM_EXPERTS), lambda i: (0, 0)),
        ],
        out_specs=pl.BlockSpec((1, NUM_EXPERTS, CHUNK), lambda i: (i, 0, 0)),
        out_shape=jax.ShapeDtypeStruct((n // CHUNK, NUM_EXPERTS, CHUNK),
                                       jnp.float32),
        scratch_shapes=[
            pltpu.VMEM((NBUF, CHUNK, HIDDEN), jnp.float32),
            pltpu.SemaphoreType.DMA((NBUF,)),
        ],
    )(x_flat, wt)

    mesh = plsc.VectorSubcoreMesh(core_axis_name="c", subcore_axis_name="s")
    topk = functools.partial(
        pl.kernel,
        mesh=mesh,
        out_type=[
            jax.ShapeDtypeStruct((n,), jnp.int32),
            jax.ShapeDtypeStruct((n,), jnp.int32),
            jax.ShapeDtypeStruct((n,), jnp.float32),
            jax.ShapeDtypeStruct((n,), jnp.float32),
        ],
        scratch_types=[
            pltpu.VMEM((NUM_EXPERTS, CHUNK), jnp.float32),
            pltpu.VMEM((CHUNK,), jnp.int32),
            pltpu.VMEM((CHUNK,), jnp.int32),
            pltpu.VMEM((CHUNK,), jnp.float32),
            pltpu.VMEM((CHUNK,), jnp.float32),
        ],
    )(_sc_topk)
    i1, i2, w1, w2 = topk(lg)

    logits = jnp.transpose(lg, (0, 2, 1)).reshape(n, NUM_EXPERTS)
    idx = jnp.stack([i1, i2], axis=-1)
    w = jnp.stack([w1, w2], axis=-1)
    return (logits, idx, w)


# speedup vs baseline: 1.7017x; 1.7017x over previous
import functools

import jax
import jax.numpy as jnp
from jax.experimental import pallas as pl
from jax.experimental.pallas import tpu as pltpu

HIDDEN = 768
NUM_EXPERTS = 8
TOP_K = 2

CHUNK = 1024
NBUF = 6


def _router_body(x_hbm, wt_ref, logits_ref, idx_ref, w_ref, xbuf, sem):
    i = pl.program_id(0)
    nchunk = pl.num_programs(0)
    slot = jax.lax.rem(i, NBUF)

    def start(c, s):
        pltpu.make_async_copy(
            x_hbm.at[pl.ds(c * CHUNK, CHUNK)], xbuf.at[s], sem.at[s]
        ).start()

    @pl.when(i == 0)
    def _():
        for b in range(NBUF):
            start(b, b)

    pltpu.make_async_copy(
        x_hbm.at[pl.ds(i * CHUNK, CHUNK)], xbuf.at[slot], sem.at[slot]
    ).wait()

    xb = xbuf[slot]
    logits = jnp.dot(xb, wt_ref[...], preferred_element_type=jnp.float32)

    lt = logits.T
    logits_ref[...] = lt
    iota = jax.lax.broadcasted_iota(jnp.int32, lt.shape, 0)
    m1 = jnp.max(lt, axis=0, keepdims=True)
    i1 = jnp.min(jnp.where(lt == m1, iota, NUM_EXPERTS), axis=0,
                 keepdims=True)
    masked = jnp.where(iota == i1, -1e30, lt)
    m2 = jnp.max(masked, axis=0, keepdims=True)
    i2 = jnp.min(jnp.where(masked == m2, iota, NUM_EXPERTS), axis=0,
                 keepdims=True)
    e = jnp.exp(m2 - m1)
    w1 = 1.0 / (1.0 + e)
    w2 = 1.0 - w1
    idx_ref[...] = jnp.concatenate([i1, i2], axis=0)
    w_ref[...] = jnp.concatenate([w1, w2], axis=0)

    @pl.when(i + NBUF < nchunk)
    def _():
        start(i + NBUF, slot)


@jax.jit
def kernel(x, W):
    b, s, h = x.shape
    n = b * s
    x_flat = x.reshape(n, h)
    wt = W.T
    logits_t, idx_t, w_t = pl.pallas_call(
        _router_body,
        grid=(n // CHUNK,),
        in_specs=[
            pl.BlockSpec(memory_space=pltpu.HBM),
            pl.BlockSpec((h, NUM_EXPERTS), lambda i: (0, 0)),
        ],
        out_specs=[
            pl.BlockSpec((NUM_EXPERTS, CHUNK), lambda i: (0, i)),
            pl.BlockSpec((TOP_K, CHUNK), lambda i: (0, i)),
            pl.BlockSpec((TOP_K, CHUNK), lambda i: (0, i)),
        ],
        out_shape=[
            jax.ShapeDtypeStruct((NUM_EXPERTS, n), jnp.float32),
            jax.ShapeDtypeStruct((TOP_K, n), jnp.int32),
            jax.ShapeDtypeStruct((TOP_K, n), jnp.float32),
        ],
        scratch_shapes=[
            pltpu.VMEM((NBUF, CHUNK, HIDDEN), jnp.float32),
            pltpu.SemaphoreType.DMA((NBUF,)),
        ],
    )(x_flat, wt)
    return (logits_t.T, idx_t.T, w_t.T)
